# baseline (speedup 1.0000x reference)
import jax
import jax.numpy as jnp
from jax.experimental import pallas as pl
from jax.experimental.pallas import tpu as pltpu

B = 16
C = 3
IMG = 384
P = 16
G = IMG // P
N = G * G
E = 4
TOPK = 2
EXPERT_DIM = 1024
HIDDEN = 1024
PATCH_DIM = C * P * P
LEAD = 4
SLOTS = LEAD + 1


def _moe_kernel(sel_ref, rw_ref, w1_ref, w2_ref, b1_ref, b2_ref, x_ref,
                out_ref, wc_scr, bc_scr, p_scr):
    i = pl.program_id(0)

    @pl.when(i < E)
    def _():
        w2 = w2_ref[0]
        wc = jnp.dot(w1_ref[0], w2, preferred_element_type=jnp.float32)
        wc_scr[i] = wc.astype(jnp.bfloat16)
        bc_scr[i] = jnp.dot(b1_ref[0], w2,
                            preferred_element_type=jnp.float32) + b2_ref[0]

    @pl.when(i < B)
    def _():
        xb = x_ref[0].astype(jnp.bfloat16)
        p_scr[i % SLOTS] = xb.reshape(C, G, P, G, P).transpose(
            1, 3, 0, 2, 4).reshape(N, PATCH_DIM)

    @pl.when(i >= LEAD)
    def _():
        j = i - LEAD
        s0 = sel_ref[j, 0]
        s1 = sel_ref[j, 1]
        w0 = rw_ref[j, 0]
        w1 = rw_ref[j, 1]

        def coef(e):
            c0 = jnp.where(s0 == e, w0, jnp.float32(0.0))
            c1 = jnp.where(s1 == e, w1, jnp.float32(0.0))
            return c0 + c1

        cs = [coef(e) for e in range(E)]
        w_eff = cs[0].astype(jnp.bfloat16) * wc_scr[0]
        for e in range(1, E):
            w_eff = w_eff + cs[e].astype(jnp.bfloat16) * wc_scr[e]
        bias = cs[0] * bc_scr[0]
        for e in range(1, E):
            bias = bias + cs[e] * bc_scr[e]

        out_ref[0] = jnp.dot(p_scr[j % SLOTS], w_eff,
                             preferred_element_type=jnp.float32) + bias


def kernel(x, selected_experts, routing_weights, W_patch, b_patch, W_proj, b_proj):
    xb = x.shape[0]
    nsteps = xb + LEAD
    e_idx = lambda i, sel, rw: (jnp.minimum(i, E - 1), 0, 0)
    out = pl.pallas_call(
        _moe_kernel,
        grid_spec=pltpu.PrefetchScalarGridSpec(
            num_scalar_prefetch=2,
            grid=(nsteps,),
            in_specs=[
                pl.BlockSpec((1, PATCH_DIM, EXPERT_DIM), e_idx),
                pl.BlockSpec((1, EXPERT_DIM, HIDDEN), e_idx),
                pl.BlockSpec((1, 1, EXPERT_DIM), e_idx),
                pl.BlockSpec((1, 1, HIDDEN), e_idx),
                pl.BlockSpec((1, C, IMG, IMG),
                             lambda i, sel, rw: (jnp.minimum(i, B - 1), 0, 0, 0)),
            ],
            out_specs=pl.BlockSpec(
                (1, N, HIDDEN),
                lambda i, sel, rw: (jnp.maximum(i - LEAD, 0), 0, 0)),
            scratch_shapes=[
                pltpu.VMEM((E, PATCH_DIM, HIDDEN), jnp.bfloat16),
                pltpu.VMEM((E, 1, HIDDEN), jnp.float32),
                pltpu.VMEM((SLOTS, N, PATCH_DIM), jnp.bfloat16),
            ],
        ),
        out_shape=jax.ShapeDtypeStruct((xb, N, HIDDEN), jnp.float32),
    )(selected_experts.astype(jnp.int32), routing_weights, W_patch, W_proj,
      b_patch.reshape(E, 1, EXPERT_DIM), b_proj.reshape(E, 1, HIDDEN), x)
    return out
